# baseline (speedup 1.0000x reference)
import functools

import jax
import jax.numpy as jnp
from jax import lax
from jax.experimental import pallas as pl
from jax.experimental.pallas import tpu as pltpu
from jax.experimental.pallas import tpu_sc as plsc

B, N, E = 4, 10000, 160000
NN = B * N
H = 2 * N
BE = B * E
NC, NS = 2, 16
NW = NC * NS
C = 80
NBUF = 5
ITERS = BE // (NW * C)
NBLK = ITERS // NBUF
EW = ITERS * C
STRIPE = 1256
STRIPE_L = H - (NS - 1) * STRIPE


@functools.lru_cache(maxsize=None)
def _make_segsum(D):
    mesh = plsc.VectorSubcoreMesh(
        core_axis_name="c", subcore_axis_name="s",
        num_cores=NC, num_subcores=NS)

    @functools.partial(
        pl.kernel,
        mesh=mesh,
        compiler_params=pltpu.CompilerParams(use_tc_tiling_on_sc=False),
        out_type=jax.ShapeDtypeStruct((NC, H, D), jnp.float32),
        scratch_types=[
            pltpu.VMEM((2, NBUF, C), jnp.int32),
            pltpu.VMEM((2, NBUF, C), jnp.int32),
            pltpu.VMEM((NBUF, C, D), jnp.float32),
            pltpu.VMEM_SHARED((H, D), jnp.float32),
            pltpu.SemaphoreType.DMA,
        ] + [pltpu.SemaphoreType.DMA] * NBUF,
    )
    def seg(table_hbm, ei_hbm, zrows_hbm, out_hbm,
            src_v, dst_v, rows_v, acc_sh, isem, *gsems):
        c = lax.axis_index("c")
        s = lax.axis_index("s")
        wid = s * NC + c
        woff = N * (wid // NS)

        @pl.when(s < NS - 1)
        def _():
            pltpu.sync_copy(zrows_hbm, acc_sh.at[pl.ds(s * STRIPE, STRIPE)])

        @pl.when(s == NS - 1)
        def _():
            pltpu.sync_copy(zrows_hbm.at[pl.ds(0, STRIPE_L)],
                            acc_sh.at[pl.ds((NS - 1) * STRIPE, STRIPE_L)])

        plsc.subcore_barrier()

        def idx_copy(fn, p, sl):
            fn(ei_hbm.at[0, wid, sl], src_v.at[p], isem)
            fn(ei_hbm.at[1, wid, sl], dst_v.at[p], isem)

        def idx_adjust(p):
            for r in range(NBUF):
                for j in range(C // 16):
                    sl = pl.ds(j * 16, 16)
                    src_v[p, r, sl] = src_v[p, r, sl] + woff
                    dst_v[p, r, sl] = dst_v[p, r, sl] + woff

        def gather(fn, p, b):
            return fn(table_hbm.at[src_v.at[p, b]], rows_v.at[b], gsems[b])

        pltpu.sync_copy(ei_hbm.at[0, wid, pl.ds(0, NBUF)], src_v.at[0])
        pltpu.sync_copy(ei_hbm.at[1, wid, pl.ds(0, NBUF)], dst_v.at[0])
        idx_adjust(0)
        for b in range(NBUF):
            gather(pltpu.async_copy, 0, b)
        idx_copy(pltpu.async_copy, 1, pl.ds(NBUF, NBUF))

        def body(t, carry):
            p = lax.rem(t, 2)
            q = lax.rem(t + 1, 2)
            sl = pl.ds((t + 1) * NBUF, NBUF)
            pltpu.make_async_copy(ei_hbm.at[0, wid, sl], src_v.at[q],
                                  isem).wait()
            pltpu.make_async_copy(ei_hbm.at[1, wid, sl], dst_v.at[q],
                                  isem).wait()
            idx_adjust(q)
            for b in range(NBUF):
                gather(pltpu.make_async_copy, p, b).wait()
                pltpu.sync_copy(rows_v.at[b], acc_sh.at[dst_v.at[p, b]],
                                add=True)
                gather(pltpu.async_copy, q, b)

            @pl.when(t + 2 < NBLK)
            def _():
                idx_copy(pltpu.async_copy, p, pl.ds((t + 2) * NBUF, NBUF))
            return carry

        lax.fori_loop(0, NBLK - 1, body, 0)
        pq = (NBLK - 1) % 2
        for b in range(NBUF):
            gather(pltpu.make_async_copy, pq, b).wait()
            pltpu.sync_copy(rows_v.at[b], acc_sh.at[dst_v.at[pq, b]], add=True)
        plsc.subcore_barrier()

        @pl.when(s < NS - 1)
        def _():
            pltpu.sync_copy(acc_sh.at[pl.ds(s * STRIPE, STRIPE)],
                            out_hbm.at[c, pl.ds(s * STRIPE, STRIPE)])

        @pl.when(s == NS - 1)
        def _():
            pltpu.sync_copy(acc_sh.at[pl.ds((NS - 1) * STRIPE, STRIPE_L)],
                            out_hbm.at[c, pl.ds((NS - 1) * STRIPE, STRIPE_L)])

    return seg


RT = 2000
GH = H // RT
GT = 2 * GH


def _mlp_a_body(x_ref, wa_ref, ba_ref, wb_ref, bb_ref, o_ref):
    t = jnp.maximum(jnp.dot(x_ref[...], wa_ref[...],
                            preferred_element_type=jnp.float32) + ba_ref[...], 0.0)
    u = jnp.dot(t, wb_ref[...], preferred_element_type=jnp.float32) + bb_ref[...]
    o_ref[...] = jnp.maximum(u, 0.0)


def _mlp_a(x0p, wa, ba, wb, bb):
    return pl.pallas_call(
        _mlp_a_body,
        grid=(GH,),
        in_specs=[
            pl.BlockSpec((RT, 16), lambda i: (i, 0)),
            pl.BlockSpec((16, 64), lambda i: (0, 0)),
            pl.BlockSpec((1, 64), lambda i: (0, 0)),
            pl.BlockSpec((64, 64), lambda i: (0, 0)),
            pl.BlockSpec((1, 64), lambda i: (0, 0)),
        ],
        out_specs=pl.BlockSpec((RT, 64), lambda i: (i, 0)),
        out_shape=jax.ShapeDtypeStruct((H, 64), jnp.float32),
    )(x0p, wa, ba, wb, bb)


def _mlp_b_body(x_ref, acc_ref, wa_ref, ba_ref, wb_ref, bb_ref, o_ref):
    xa = x_ref[...] + acc_ref[0] + acc_ref[1]
    t = jnp.maximum(jnp.dot(xa, wa_ref[...],
                            preferred_element_type=jnp.float32) + ba_ref[...], 0.0)
    u = jnp.dot(t, wb_ref[...], preferred_element_type=jnp.float32) + bb_ref[...]
    o_ref[...] = jnp.maximum(u, 0.0)


def _mlp_b(x0p, acc1, wa, ba, wb, bb):
    return pl.pallas_call(
        _mlp_b_body,
        grid=(GH,),
        in_specs=[
            pl.BlockSpec((RT, 16), lambda i: (i + GH, 0)),
            pl.BlockSpec((NC, RT, 16), lambda i: (0, i, 0)),
            pl.BlockSpec((16, 64), lambda i: (0, 0)),
            pl.BlockSpec((1, 64), lambda i: (0, 0)),
            pl.BlockSpec((64, 64), lambda i: (0, 0)),
            pl.BlockSpec((1, 64), lambda i: (0, 0)),
        ],
        out_specs=pl.BlockSpec((RT, 64), lambda i: (i, 0)),
        out_shape=jax.ShapeDtypeStruct((H, 64), jnp.float32),
    )(x0p, acc1, wa, ba, wb, bb)


def _head_body(ha_ref, hb_ref, acc_ref, wa_ref, ba_ref, wb_ref, bb_ref,
               wm_ref, bm_ref, wo_ref, bo_ref, o_ref, pool_ref):
    i = pl.program_id(0)
    m = jnp.where(i >= GH, 1.0, 0.0).astype(jnp.float32)
    xa = (1.0 - m) * ha_ref[...] + m * (hb_ref[...]
                                        + acc_ref[0] + acc_ref[1])
    t = jnp.maximum(jnp.dot(xa, wa_ref[...],
                            preferred_element_type=jnp.float32) + ba_ref[...], 0.0)
    u = jnp.dot(t, wb_ref[...], preferred_element_type=jnp.float32) + bb_ref[...]
    h2 = jnp.maximum(u, 0.0)

    @pl.when(i == 0)
    def _():
        pool_ref[...] = jnp.zeros_like(pool_ref)

    b = i // (GT // B)
    onehot = (lax.broadcasted_iota(jnp.int32, (B, 1), 0) == b).astype(jnp.float32)
    pool_ref[...] += onehot * jnp.sum(h2, axis=0, keepdims=True)

    @pl.when(i == GT - 1)
    def _():
        g = jnp.maximum(jnp.dot(pool_ref[...], wm_ref[...],
                                preferred_element_type=jnp.float32) + bm_ref[...], 0.0)
        z = jnp.dot(g, wo_ref[...], preferred_element_type=jnp.float32) + bo_ref[...]
        o_ref[...] = 1.0 / (1.0 + jnp.exp(-z))


def _mlp2_pool_head(h1a, h1b, acc2, wa, ba, wb, bb, wm, bm, wo, bo):
    return pl.pallas_call(
        _head_body,
        grid=(GT,),
        in_specs=[
            pl.BlockSpec((RT, 64), lambda i: (jnp.minimum(i, GH - 1), 0)),
            pl.BlockSpec((RT, 64), lambda i: (jnp.maximum(i - GH, 0), 0)),
            pl.BlockSpec((NC, RT, 64), lambda i: (0, jnp.maximum(i - GH, 0), 0)),
            pl.BlockSpec((64, 64), lambda i: (0, 0)),
            pl.BlockSpec((1, 64), lambda i: (0, 0)),
            pl.BlockSpec((64, 64), lambda i: (0, 0)),
            pl.BlockSpec((1, 64), lambda i: (0, 0)),
            pl.BlockSpec((64, 64), lambda i: (0, 0)),
            pl.BlockSpec((1, 64), lambda i: (0, 0)),
            pl.BlockSpec((64, N), lambda i: (0, 0)),
            pl.BlockSpec((1, N), lambda i: (0, 0)),
        ],
        out_specs=pl.BlockSpec((B, N), lambda i: (0, 0)),
        out_shape=jax.ShapeDtypeStruct((B, N), jnp.float32),
        scratch_shapes=[pltpu.VMEM((B, 64), jnp.float32)],
    )(h1a, h1b, acc2, wa, ba, wb, bb, wm, bm, wo, bo)


def kernel(actions, node_features, edge_index, W0a, b0a, W0b, b0b,
           W1a, b1a, W1b, b1b, Wm, bm, Wo, bo):
    nf = node_features.reshape(B, N).astype(jnp.float32)
    x0 = jnp.stack((actions[:, :, 0], actions[:, :, 1], nf), axis=2).reshape(NN, 3)
    x0p = jnp.pad(x0, ((0, 0), (0, 13)))

    ei = edge_index.reshape(2, NW, ITERS, C)

    z16 = jnp.zeros((STRIPE, 16), jnp.float32)
    z64 = jnp.zeros((STRIPE, 64), jnp.float32)

    W0a_p = jnp.pad(W0a, ((0, 13), (0, 0)))
    acc1 = _make_segsum(16)(x0p, ei, z16)
    h1a = _mlp_a(x0p, W0a_p, b0a.reshape(1, 64), W0b, b0b.reshape(1, 64))
    h1a_q, acc1 = lax.optimization_barrier((h1a, acc1))
    acc2 = _make_segsum(64)(h1a_q, ei, z64)
    h1b = _mlp_b(x0p, acc1, W0a_p, b0a.reshape(1, 64), W0b, b0b.reshape(1, 64))
    out = _mlp2_pool_head(h1a, h1b, acc2, W1a, b1a.reshape(1, 64), W1b,
                          b1b.reshape(1, 64), Wm, bm.reshape(1, 64),
                          Wo, bo.reshape(1, N))
    return out


# speedup vs baseline: 1.0269x; 1.0269x over previous
import functools

import jax
import jax.numpy as jnp
from jax import lax
from jax.experimental import pallas as pl
from jax.experimental.pallas import tpu as pltpu
from jax.experimental.pallas import tpu_sc as plsc

B, N, E = 4, 10000, 160000
NN = B * N
H = 2 * N
BE = B * E
NC, NS = 2, 16
NW = NC * NS
C = 80
NBUF = 5
ITERS = BE // (NW * C)
NBLK = ITERS // NBUF
EW = ITERS * C
STRIPE = 1256
STRIPE_L = H - (NS - 1) * STRIPE


@functools.lru_cache(maxsize=None)
def _make_segsum(D):
    mesh = plsc.VectorSubcoreMesh(
        core_axis_name="c", subcore_axis_name="s",
        num_cores=NC, num_subcores=NS)

    @functools.partial(
        pl.kernel,
        mesh=mesh,
        compiler_params=pltpu.CompilerParams(use_tc_tiling_on_sc=False),
        out_type=jax.ShapeDtypeStruct((NC, H, D), jnp.float32),
        scratch_types=[
            pltpu.VMEM((2, NBUF, C), jnp.int32),
            pltpu.VMEM((2, NBUF, C), jnp.int32),
            pltpu.VMEM((NBUF, C, D), jnp.float32),
            pltpu.VMEM_SHARED((H, D), jnp.float32),
            pltpu.SemaphoreType.DMA,
        ] + [pltpu.SemaphoreType.DMA] * NBUF,
    )
    def seg(table_hbm, ei_hbm, zrows_hbm, out_hbm,
            src_v, dst_v, rows_v, acc_sh, isem, *gsems):
        c = lax.axis_index("c")
        s = lax.axis_index("s")
        wid = s * NC + c
        woff = N * (wid // NS)

        @pl.when(s < NS - 1)
        def _():
            pltpu.sync_copy(zrows_hbm, acc_sh.at[pl.ds(s * STRIPE, STRIPE)])

        @pl.when(s == NS - 1)
        def _():
            pltpu.sync_copy(zrows_hbm.at[pl.ds(0, STRIPE_L)],
                            acc_sh.at[pl.ds((NS - 1) * STRIPE, STRIPE_L)])

        plsc.subcore_barrier()

        def idx_copy(fn, p, sl):
            fn(ei_hbm.at[0, wid, sl], src_v.at[p], isem)
            fn(ei_hbm.at[1, wid, sl], dst_v.at[p], isem)

        def idx_adjust(p):
            for r in range(NBUF):
                for j in range(C // 16):
                    sl = pl.ds(j * 16, 16)
                    src_v[p, r, sl] = src_v[p, r, sl] + woff
                    dst_v[p, r, sl] = dst_v[p, r, sl] + woff

        def gather(fn, p, b):
            return fn(table_hbm.at[src_v.at[p, b]], rows_v.at[b], gsems[b])

        pltpu.sync_copy(ei_hbm.at[0, wid, pl.ds(0, NBUF)], src_v.at[0])
        pltpu.sync_copy(ei_hbm.at[1, wid, pl.ds(0, NBUF)], dst_v.at[0])
        idx_adjust(0)
        for b in range(NBUF):
            gather(pltpu.async_copy, 0, b)
        idx_copy(pltpu.async_copy, 1, pl.ds(NBUF, NBUF))

        def body(t, carry):
            p = lax.rem(t, 2)
            q = lax.rem(t + 1, 2)
            sl = pl.ds((t + 1) * NBUF, NBUF)
            pltpu.make_async_copy(ei_hbm.at[0, wid, sl], src_v.at[q],
                                  isem).wait()
            pltpu.make_async_copy(ei_hbm.at[1, wid, sl], dst_v.at[q],
                                  isem).wait()
            idx_adjust(q)
            for b in range(NBUF):
                gather(pltpu.make_async_copy, p, b).wait()
                pltpu.sync_copy(rows_v.at[b], acc_sh.at[dst_v.at[p, b]],
                                add=True)
                gather(pltpu.async_copy, q, b)

            @pl.when(t + 2 < NBLK)
            def _():
                idx_copy(pltpu.async_copy, p, pl.ds((t + 2) * NBUF, NBUF))
            return carry

        lax.fori_loop(0, NBLK - 1, body, 0)
        pq = (NBLK - 1) % 2
        for b in range(NBUF):
            gather(pltpu.make_async_copy, pq, b).wait()
            pltpu.sync_copy(rows_v.at[b], acc_sh.at[dst_v.at[pq, b]], add=True)
        plsc.subcore_barrier()

        @pl.when(s < NS - 1)
        def _():
            pltpu.sync_copy(acc_sh.at[pl.ds(s * STRIPE, STRIPE)],
                            out_hbm.at[c, pl.ds(s * STRIPE, STRIPE)])

        @pl.when(s == NS - 1)
        def _():
            pltpu.sync_copy(acc_sh.at[pl.ds((NS - 1) * STRIPE, STRIPE_L)],
                            out_hbm.at[c, pl.ds((NS - 1) * STRIPE, STRIPE_L)])

    return seg


RT = 2000
GH = H // RT
GT = 2 * GH


def _mlp_a_body(x_ref, wa_ref, ba_ref, wb_ref, bb_ref, o_ref):
    t = jnp.maximum(jnp.dot(x_ref[...], wa_ref[...],
                            preferred_element_type=jnp.float32) + ba_ref[...], 0.0)
    u = jnp.dot(t, wb_ref[...], preferred_element_type=jnp.float32) + bb_ref[...]
    o_ref[...] = jnp.maximum(u, 0.0)


def _mlp_a(x0p, wa, ba, wb, bb):
    return pl.pallas_call(
        _mlp_a_body,
        grid=(GH,),
        in_specs=[
            pl.BlockSpec((RT, 16), lambda i: (i, 0)),
            pl.BlockSpec((16, 64), lambda i: (0, 0)),
            pl.BlockSpec((1, 64), lambda i: (0, 0)),
            pl.BlockSpec((64, 64), lambda i: (0, 0)),
            pl.BlockSpec((1, 64), lambda i: (0, 0)),
        ],
        out_specs=pl.BlockSpec((RT, 64), lambda i: (i, 0)),
        out_shape=jax.ShapeDtypeStruct((H, 64), jnp.float32),
    )(x0p, wa, ba, wb, bb)


def _mlp_b_body(x_ref, acc_ref, wa_ref, ba_ref, wb_ref, bb_ref, o_ref):
    xa = x_ref[...] + acc_ref[0] + acc_ref[1]
    t = jnp.maximum(jnp.dot(xa, wa_ref[...],
                            preferred_element_type=jnp.float32) + ba_ref[...], 0.0)
    u = jnp.dot(t, wb_ref[...], preferred_element_type=jnp.float32) + bb_ref[...]
    o_ref[...] = jnp.maximum(u, 0.0)


def _mlp_b(x0p, acc1, wa, ba, wb, bb):
    return pl.pallas_call(
        _mlp_b_body,
        grid=(GH,),
        in_specs=[
            pl.BlockSpec((RT, 16), lambda i: (i + GH, 0)),
            pl.BlockSpec((NC, RT, 16), lambda i: (0, i, 0)),
            pl.BlockSpec((16, 64), lambda i: (0, 0)),
            pl.BlockSpec((1, 64), lambda i: (0, 0)),
            pl.BlockSpec((64, 64), lambda i: (0, 0)),
            pl.BlockSpec((1, 64), lambda i: (0, 0)),
        ],
        out_specs=pl.BlockSpec((RT, 64), lambda i: (i, 0)),
        out_shape=jax.ShapeDtypeStruct((H, 64), jnp.float32),
    )(x0p, acc1, wa, ba, wb, bb)


def _head_body(ha_ref, hb_ref, acc_ref, wa_ref, ba_ref, wb_ref, bb_ref,
               wm_ref, bm_ref, wo_ref, bo_ref, o_ref, pool_ref):
    i = pl.program_id(0)
    m = jnp.where(i >= GH, 1.0, 0.0).astype(jnp.float32)
    xa = (1.0 - m) * ha_ref[...] + m * (hb_ref[...]
                                        + acc_ref[0] + acc_ref[1])
    t = jnp.maximum(jnp.dot(xa, wa_ref[...],
                            preferred_element_type=jnp.float32) + ba_ref[...], 0.0)
    u = jnp.dot(t, wb_ref[...], preferred_element_type=jnp.float32) + bb_ref[...]
    h2 = jnp.maximum(u, 0.0)

    @pl.when(i == 0)
    def _():
        pool_ref[...] = jnp.zeros_like(pool_ref)

    b = i // (GT // B)
    onehot = (lax.broadcasted_iota(jnp.int32, (B, 1), 0) == b).astype(jnp.float32)
    pool_ref[...] += onehot * jnp.sum(h2, axis=0, keepdims=True)

    @pl.when(i == GT - 1)
    def _():
        g = jnp.maximum(jnp.dot(pool_ref[...], wm_ref[...],
                                preferred_element_type=jnp.float32) + bm_ref[...], 0.0)
        z = jnp.dot(g, wo_ref[...], preferred_element_type=jnp.float32) + bo_ref[...]
        o_ref[...] = 1.0 / (1.0 + jnp.exp(-z))


def _mlp2_pool_head(h1a, h1b, acc2, wa, ba, wb, bb, wm, bm, wo, bo):
    return pl.pallas_call(
        _head_body,
        grid=(GT,),
        in_specs=[
            pl.BlockSpec((RT, 64), lambda i: (jnp.minimum(i, GH - 1), 0)),
            pl.BlockSpec((RT, 64), lambda i: (jnp.maximum(i - GH, 0), 0)),
            pl.BlockSpec((NC, RT, 64), lambda i: (0, jnp.maximum(i - GH, 0), 0)),
            pl.BlockSpec((64, 64), lambda i: (0, 0)),
            pl.BlockSpec((1, 64), lambda i: (0, 0)),
            pl.BlockSpec((64, 64), lambda i: (0, 0)),
            pl.BlockSpec((1, 64), lambda i: (0, 0)),
            pl.BlockSpec((64, 64), lambda i: (0, 0)),
            pl.BlockSpec((1, 64), lambda i: (0, 0)),
            pl.BlockSpec((64, N), lambda i: (0, 0)),
            pl.BlockSpec((1, N), lambda i: (0, 0)),
        ],
        out_specs=pl.BlockSpec((B, N), lambda i: (0, 0)),
        out_shape=jax.ShapeDtypeStruct((B, N), jnp.float32),
        scratch_shapes=[pltpu.VMEM((B, 64), jnp.float32)],
    )(h1a, h1b, acc2, wa, ba, wb, bb, wm, bm, wo, bo)


def kernel(actions, node_features, edge_index, W0a, b0a, W0b, b0b,
           W1a, b1a, W1b, b1b, Wm, bm, Wo, bo):
    x0p = jnp.concatenate(
        [actions.reshape(NN, 2).astype(jnp.float32),
         node_features.reshape(NN, 1).astype(jnp.float32),
         jnp.zeros((NN, 13), jnp.float32)], axis=1)

    ei = edge_index.reshape(2, NW, ITERS, C)

    z16 = jnp.zeros((STRIPE, 16), jnp.float32)
    z64 = jnp.zeros((STRIPE, 64), jnp.float32)

    W0a_p = jnp.pad(W0a, ((0, 13), (0, 0)))
    acc1 = _make_segsum(16)(x0p, ei, z16)
    h1a = _mlp_a(x0p, W0a_p, b0a.reshape(1, 64), W0b, b0b.reshape(1, 64))
    h1a_q, acc1 = lax.optimization_barrier((h1a, acc1))
    acc2 = _make_segsum(64)(h1a_q, ei, z64)
    h1b = _mlp_b(x0p, acc1, W0a_p, b0a.reshape(1, 64), W0b, b0b.reshape(1, 64))
    out = _mlp2_pool_head(h1a, h1b, acc2, W1a, b1a.reshape(1, 64), W1b,
                          b1b.reshape(1, 64), Wm, bm.reshape(1, 64),
                          Wo, bo.reshape(1, N))
    return out


# speedup vs baseline: 1.0470x; 1.0196x over previous
import functools

import jax
import jax.numpy as jnp
from jax import lax
from jax.experimental import pallas as pl
from jax.experimental.pallas import tpu as pltpu
from jax.experimental.pallas import tpu_sc as plsc

B, N, E = 4, 10000, 160000
NN = B * N
H = 2 * N
BE = B * E
NC, NS = 2, 16
NW = NC * NS
C = 80
NBUF = 5
ITERS = BE // (NW * C)
NBLK = ITERS // NBUF
EW = ITERS * C
STRIPE = 1256
STRIPE_L = H - (NS - 1) * STRIPE


@functools.lru_cache(maxsize=None)
def _make_segsum(D):
    mesh = plsc.VectorSubcoreMesh(
        core_axis_name="c", subcore_axis_name="s",
        num_cores=NC, num_subcores=NS)

    @functools.partial(
        pl.kernel,
        mesh=mesh,
        compiler_params=pltpu.CompilerParams(use_tc_tiling_on_sc=False),
        out_type=jax.ShapeDtypeStruct((NC, H, D), jnp.float32),
        scratch_types=[
            pltpu.VMEM((2, NBUF, C), jnp.int32),
            pltpu.VMEM((2, NBUF, C), jnp.int32),
            pltpu.VMEM((NBUF, C, D), jnp.float32),
            pltpu.VMEM_SHARED((H, D), jnp.float32),
            pltpu.SemaphoreType.DMA,
        ] + [pltpu.SemaphoreType.DMA] * (2 * NBUF),
    )
    def seg(table_hbm, ei_hbm, zrows_hbm, out_hbm,
            src_v, dst_v, rows_v, acc_sh, isem, *gssems):
        gsems, ssems = gssems[:NBUF], gssems[NBUF:]
        c = lax.axis_index("c")
        s = lax.axis_index("s")
        wid = s * NC + c
        woff = N * (wid // NS)

        @pl.when(s < NS - 1)
        def _():
            pltpu.sync_copy(zrows_hbm, acc_sh.at[pl.ds(s * STRIPE, STRIPE)])

        @pl.when(s == NS - 1)
        def _():
            pltpu.sync_copy(zrows_hbm.at[pl.ds(0, STRIPE_L)],
                            acc_sh.at[pl.ds((NS - 1) * STRIPE, STRIPE_L)])

        plsc.subcore_barrier()

        def idx_copy(fn, p, sl):
            fn(ei_hbm.at[0, wid, sl], src_v.at[p], isem)
            fn(ei_hbm.at[1, wid, sl], dst_v.at[p], isem)

        def idx_adjust(p):
            for r in range(NBUF):
                for j in range(C // 16):
                    sl = pl.ds(j * 16, 16)
                    src_v[p, r, sl] = src_v[p, r, sl] + woff
                    dst_v[p, r, sl] = dst_v[p, r, sl] + woff

        def gather(fn, p, b):
            return fn(table_hbm.at[src_v.at[p, b]], rows_v.at[b], gsems[b])

        def scatter(fn, p, b, **kw):
            return fn(rows_v.at[b], acc_sh.at[dst_v.at[p, b]], ssems[b], **kw)

        pltpu.sync_copy(ei_hbm.at[0, wid, pl.ds(0, NBUF)], src_v.at[0])
        pltpu.sync_copy(ei_hbm.at[1, wid, pl.ds(0, NBUF)], dst_v.at[0])
        idx_adjust(0)
        for b in range(NBUF):
            gather(pltpu.async_copy, 0, b)
        idx_copy(pltpu.async_copy, 1, pl.ds(NBUF, NBUF))

        def body(t, carry):
            p = lax.rem(t, 2)
            q = lax.rem(t + 1, 2)
            sl = pl.ds((t + 1) * NBUF, NBUF)
            pltpu.make_async_copy(ei_hbm.at[0, wid, sl], src_v.at[q],
                                  isem).wait()
            pltpu.make_async_copy(ei_hbm.at[1, wid, sl], dst_v.at[q],
                                  isem).wait()
            idx_adjust(q)
            for b in range(NBUF):
                gather(pltpu.make_async_copy, p, b).wait()
                scatter(pltpu.async_copy, p, b, add=True)
            for b in range(NBUF):
                scatter(pltpu.make_async_copy, p, b).wait()
                gather(pltpu.async_copy, q, b)

            @pl.when(t + 2 < NBLK)
            def _():
                idx_copy(pltpu.async_copy, p, pl.ds((t + 2) * NBUF, NBUF))
            return carry

        lax.fori_loop(0, NBLK - 1, body, 0)
        pq = (NBLK - 1) % 2
        for b in range(NBUF):
            gather(pltpu.make_async_copy, pq, b).wait()
            scatter(pltpu.async_copy, pq, b, add=True)
        for b in range(NBUF):
            scatter(pltpu.make_async_copy, pq, b).wait()
        plsc.subcore_barrier()

        @pl.when(s < NS - 1)
        def _():
            pltpu.sync_copy(acc_sh.at[pl.ds(s * STRIPE, STRIPE)],
                            out_hbm.at[c, pl.ds(s * STRIPE, STRIPE)])

        @pl.when(s == NS - 1)
        def _():
            pltpu.sync_copy(acc_sh.at[pl.ds((NS - 1) * STRIPE, STRIPE_L)],
                            out_hbm.at[c, pl.ds((NS - 1) * STRIPE, STRIPE_L)])

    return seg


RT = 2000
GH = H // RT
GT = 2 * GH


def _mlp_a_body(x_ref, wa_ref, ba_ref, wb_ref, bb_ref, o_ref):
    t = jnp.maximum(jnp.dot(x_ref[...], wa_ref[...],
                            preferred_element_type=jnp.float32) + ba_ref[...], 0.0)
    u = jnp.dot(t, wb_ref[...], preferred_element_type=jnp.float32) + bb_ref[...]
    o_ref[...] = jnp.maximum(u, 0.0)


def _mlp_a(x0p, wa, ba, wb, bb):
    return pl.pallas_call(
        _mlp_a_body,
        grid=(GH,),
        in_specs=[
            pl.BlockSpec((RT, 16), lambda i: (i, 0)),
            pl.BlockSpec((16, 64), lambda i: (0, 0)),
            pl.BlockSpec((1, 64), lambda i: (0, 0)),
            pl.BlockSpec((64, 64), lambda i: (0, 0)),
            pl.BlockSpec((1, 64), lambda i: (0, 0)),
        ],
        out_specs=pl.BlockSpec((RT, 64), lambda i: (i, 0)),
        out_shape=jax.ShapeDtypeStruct((H, 64), jnp.float32),
    )(x0p, wa, ba, wb, bb)


def _mlp_b_body(x_ref, acc_ref, wa_ref, ba_ref, wb_ref, bb_ref, o_ref):
    xa = x_ref[...] + acc_ref[0] + acc_ref[1]
    t = jnp.maximum(jnp.dot(xa, wa_ref[...],
                            preferred_element_type=jnp.float32) + ba_ref[...], 0.0)
    u = jnp.dot(t, wb_ref[...], preferred_element_type=jnp.float32) + bb_ref[...]
    o_ref[...] = jnp.maximum(u, 0.0)


def _mlp_b(x0p, acc1, wa, ba, wb, bb):
    return pl.pallas_call(
        _mlp_b_body,
        grid=(GH,),
        in_specs=[
            pl.BlockSpec((RT, 16), lambda i: (i + GH, 0)),
            pl.BlockSpec((NC, RT, 16), lambda i: (0, i, 0)),
            pl.BlockSpec((16, 64), lambda i: (0, 0)),
            pl.BlockSpec((1, 64), lambda i: (0, 0)),
            pl.BlockSpec((64, 64), lambda i: (0, 0)),
            pl.BlockSpec((1, 64), lambda i: (0, 0)),
        ],
        out_specs=pl.BlockSpec((RT, 64), lambda i: (i, 0)),
        out_shape=jax.ShapeDtypeStruct((H, 64), jnp.float32),
    )(x0p, acc1, wa, ba, wb, bb)


def _head_body(ha_ref, hb_ref, acc_ref, wa_ref, ba_ref, wb_ref, bb_ref,
               wm_ref, bm_ref, wo_ref, bo_ref, o_ref, pool_ref):
    i = pl.program_id(0)
    m = jnp.where(i >= GH, 1.0, 0.0).astype(jnp.float32)
    xa = (1.0 - m) * ha_ref[...] + m * (hb_ref[...]
                                        + acc_ref[0] + acc_ref[1])
    t = jnp.maximum(jnp.dot(xa, wa_ref[...],
                            preferred_element_type=jnp.float32) + ba_ref[...], 0.0)
    u = jnp.dot(t, wb_ref[...], preferred_element_type=jnp.float32) + bb_ref[...]
    h2 = jnp.maximum(u, 0.0)

    @pl.when(i == 0)
    def _():
        pool_ref[...] = jnp.zeros_like(pool_ref)

    b = i // (GT // B)
    onehot = (lax.broadcasted_iota(jnp.int32, (B, 1), 0) == b).astype(jnp.float32)
    pool_ref[...] += onehot * jnp.sum(h2, axis=0, keepdims=True)

    @pl.when(i == GT - 1)
    def _():
        g = jnp.maximum(jnp.dot(pool_ref[...], wm_ref[...],
                                preferred_element_type=jnp.float32) + bm_ref[...], 0.0)
        z = jnp.dot(g, wo_ref[...], preferred_element_type=jnp.float32) + bo_ref[...]
        o_ref[...] = 1.0 / (1.0 + jnp.exp(-z))


def _mlp2_pool_head(h1a, h1b, acc2, wa, ba, wb, bb, wm, bm, wo, bo):
    return pl.pallas_call(
        _head_body,
        grid=(GT,),
        in_specs=[
            pl.BlockSpec((RT, 64), lambda i: (jnp.minimum(i, GH - 1), 0)),
            pl.BlockSpec((RT, 64), lambda i: (jnp.maximum(i - GH, 0), 0)),
            pl.BlockSpec((NC, RT, 64), lambda i: (0, jnp.maximum(i - GH, 0), 0)),
            pl.BlockSpec((64, 64), lambda i: (0, 0)),
            pl.BlockSpec((1, 64), lambda i: (0, 0)),
            pl.BlockSpec((64, 64), lambda i: (0, 0)),
            pl.BlockSpec((1, 64), lambda i: (0, 0)),
            pl.BlockSpec((64, 64), lambda i: (0, 0)),
            pl.BlockSpec((1, 64), lambda i: (0, 0)),
            pl.BlockSpec((64, N), lambda i: (0, 0)),
            pl.BlockSpec((1, N), lambda i: (0, 0)),
        ],
        out_specs=pl.BlockSpec((B, N), lambda i: (0, 0)),
        out_shape=jax.ShapeDtypeStruct((B, N), jnp.float32),
        scratch_shapes=[pltpu.VMEM((B, 64), jnp.float32)],
    )(h1a, h1b, acc2, wa, ba, wb, bb, wm, bm, wo, bo)


def kernel(actions, node_features, edge_index, W0a, b0a, W0b, b0b,
           W1a, b1a, W1b, b1b, Wm, bm, Wo, bo):
    x0p = jnp.concatenate(
        [actions.reshape(NN, 2).astype(jnp.float32),
         node_features.reshape(NN, 1).astype(jnp.float32),
         jnp.zeros((NN, 13), jnp.float32)], axis=1)

    ei = edge_index.reshape(2, NW, ITERS, C)

    z16 = jnp.zeros((STRIPE, 16), jnp.float32)
    z64 = jnp.zeros((STRIPE, 64), jnp.float32)

    W0a_p = jnp.pad(W0a, ((0, 13), (0, 0)))
    acc1 = _make_segsum(16)(x0p, ei, z16)
    h1a = _mlp_a(x0p, W0a_p, b0a.reshape(1, 64), W0b, b0b.reshape(1, 64))
    h1a_q, acc1 = lax.optimization_barrier((h1a, acc1))
    acc2 = _make_segsum(64)(h1a_q, ei, z64)
    h1b = _mlp_b(x0p, acc1, W0a_p, b0a.reshape(1, 64), W0b, b0b.reshape(1, 64))
    out = _mlp2_pool_head(h1a, h1b, acc2, W1a, b1a.reshape(1, 64), W1b,
                          b1b.reshape(1, 64), Wm, bm.reshape(1, 64),
                          Wo, bo.reshape(1, N))
    return out
